# baseline (speedup 1.0000x reference)
import functools

import jax
import jax.numpy as jnp
from jax import lax
from jax.experimental import pallas as pl
from jax.experimental.pallas import tpu as pltpu
from jax.experimental.pallas import tpu_sc as plsc

_E = 8
_NG = 4
_TK = 2
_RSF = 2.5
_BM = 256
_BT = 512

_NEG_INF = float("-inf")


def _router_body(lg_ref, b_ref, comb_ref):
    logits = lg_ref[...]
    s = jax.nn.sigmoid(logits)
    sc = s + b_ref[...]

    bt = logits.shape[0]
    gs = jnp.concatenate(
        [sc[:, 2 * g:2 * g + 1] + sc[:, 2 * g + 1:2 * g + 2]
         for g in range(_NG)], axis=1)

    i4 = lax.broadcasted_iota(jnp.int32, (bt, _NG), 1)
    m1 = jnp.max(gs, axis=1, keepdims=True)
    g1 = jnp.min(jnp.where(gs == m1, i4, _NG), axis=1, keepdims=True)
    gs2 = jnp.where(i4 == g1, _NEG_INF, gs)
    m2 = jnp.max(gs2, axis=1, keepdims=True)
    g2 = jnp.min(jnp.where(gs2 == m2, i4, _NG), axis=1, keepdims=True)
    gmask = ((i4 == g1) | (i4 == g2)).astype(jnp.float32)
    smask = jnp.concatenate(
        [gmask[:, e // 2:e // 2 + 1] for e in range(_E)], axis=1)

    sfc = jnp.where(smask > 0, sc, 0.0)
    i8 = lax.broadcasted_iota(jnp.int32, (bt, _E), 1)
    e_m1 = jnp.max(sfc, axis=1, keepdims=True)
    e1 = jnp.min(jnp.where(sfc == e_m1, i8, _E), axis=1, keepdims=True)
    sfc2 = jnp.where(i8 == e1, _NEG_INF, sfc)
    e_m2 = jnp.max(sfc2, axis=1, keepdims=True)
    e2 = jnp.min(jnp.where(sfc2 == e_m2, i8, _E), axis=1, keepdims=True)

    sel1 = (i8 == e1)
    sel2 = (i8 == e2)
    w1 = jnp.sum(jnp.where(sel1, s, 0.0), axis=1, keepdims=True)
    w2 = jnp.sum(jnp.where(sel2, s, 0.0), axis=1, keepdims=True)
    scale = _RSF / (w1 + w2 + 1e-20)
    comb_ref[...] = (jnp.where(sel1, w1, 0.0) + jnp.where(sel2, w2, 0.0)) * scale


def _router(logits, bias):
    t = logits.shape[0]
    return pl.pallas_call(
        _router_body,
        grid=(t // _BT,),
        in_specs=[
            pl.BlockSpec((_BT, _E), lambda i: (i, 0)),
            pl.BlockSpec((1, _E), lambda i: (0, 0)),
        ],
        out_specs=pl.BlockSpec((_BT, _E), lambda i: (i, 0)),
        out_shape=jax.ShapeDtypeStruct((t, _E), jnp.float32),
    )(logits, bias.reshape(1, _E))


def _dispatch_indices(comb, t):
    mask = comb > 0.0
    mi = mask.astype(jnp.int32)
    cnt = jnp.sum(mi, axis=0)
    rank = jnp.cumsum(mi, axis=0) - mi
    padded = ((cnt + _BM - 1) // _BM) * _BM
    base = jnp.concatenate([jnp.zeros((1,), jnp.int32),
                            jnp.cumsum(padded)[:-1].astype(jnp.int32)])
    pt = _TK * t + _E * _BM
    pos = base[None, :] + rank
    posm = jnp.where(mask, pos, pt)
    tok = jnp.broadcast_to(jnp.arange(t, dtype=jnp.int32)[:, None], (t, _E))
    tok_for_pos = (jnp.arange(pt + 1, dtype=jnp.int32) % t).at[
        posm.reshape(-1)].set(tok.reshape(-1))[:pt]
    w_for_pos = jnp.zeros((pt + 1,), jnp.float32).at[posm.reshape(-1)].set(
        comb.reshape(-1))[:pt]
    bstart = base // _BM
    blk = jnp.arange(pt // _BM, dtype=jnp.int32)
    nbe = jnp.sum((blk[:, None] >= bstart[None, :]).astype(jnp.int32),
                  axis=1) - 1
    e1 = jnp.argmax(mi, axis=1)
    e2 = (_E - 1) - jnp.argmax(mi[:, ::-1], axis=1)
    ar = jnp.arange(t)
    pa = pos[ar, e1].astype(jnp.int32)
    pb = pos[ar, e2].astype(jnp.int32)
    return tok_for_pos, w_for_pos, nbe, pa, pb, pt


def _dispatch(x, tok_for_pos, pt):
    t, h = x.shape
    info = plsc.get_sparse_core_info()
    nw = info.num_cores * info.num_subcores
    rows_per_w = pt // nw
    ch = 16
    nbuf = 4
    n_ch = rows_per_w // ch
    mesh = plsc.VectorSubcoreMesh(core_axis_name="c", subcore_axis_name="s")

    @functools.partial(
        pl.kernel, mesh=mesh,
        out_type=jax.ShapeDtypeStruct((pt, h), jnp.float32),
        scratch_types=[
            pltpu.VMEM((nbuf, ch), jnp.int32),
            [pltpu.VMEM((ch, h), jnp.float32) for _ in range(nbuf)],
            [pltpu.SemaphoreType.DMA for _ in range(nbuf)],
        ],
    )
    def k(x_hbm, tok_hbm, out_hbm, idx_v, rows, sems):
        wid = lax.axis_index("s") * info.num_cores + lax.axis_index("c")
        base = wid * rows_per_w
        d = [None] * nbuf

        def start(c, b):
            pltpu.sync_copy(tok_hbm.at[pl.ds(base + c * ch, ch)], idx_v.at[b])
            d[b] = pltpu.async_copy(x_hbm.at[idx_v.at[b]], rows[b], sems[b])

        for b in range(nbuf - 1):
            start(b, b)
        for c in range(n_ch):
            b = c % nbuf
            if c + nbuf - 1 < n_ch:
                start(c + nbuf - 1, (c + nbuf - 1) % nbuf)
            d[b].wait()
            pltpu.sync_copy(rows[b], out_hbm.at[pl.ds(base + c * ch, ch)])

    return k(x, tok_for_pos)


def _gmm_body(nbe_ref, xd_ref, wg_ref, wu_ref, wd_ref, w_ref, ye_ref):
    xb = xd_ref[...].astype(jnp.bfloat16)
    wg = wg_ref[0].astype(jnp.bfloat16)
    wu = wu_ref[0].astype(jnp.bfloat16)
    g = lax.dot_general(xb, wg, (((1,), (1,)), ((), ())),
                        preferred_element_type=jnp.float32)
    u = lax.dot_general(xb, wu, (((1,), (1,)), ((), ())),
                        preferred_element_type=jnp.float32)
    a = (g * jax.nn.sigmoid(g) * u).astype(jnp.bfloat16)
    y = lax.dot_general(a, wd_ref[0].astype(jnp.bfloat16),
                        (((1,), (1,)), ((), ())),
                        preferred_element_type=jnp.float32)
    ye_ref[...] = y * w_ref[...]


def _gmm(xd, Wg, Wu, Wd, w_for_pos, nbe, pt):
    h = xd.shape[1]
    i_dim = Wg.shape[1]
    nblk = pt // _BM
    grid_spec = pltpu.PrefetchScalarGridSpec(
        num_scalar_prefetch=1,
        grid=(nblk,),
        in_specs=[
            pl.BlockSpec((_BM, h), lambda i, nbe: (i, 0)),
            pl.BlockSpec((1, i_dim, h), lambda i, nbe: (nbe[i], 0, 0)),
            pl.BlockSpec((1, i_dim, h), lambda i, nbe: (nbe[i], 0, 0)),
            pl.BlockSpec((1, h, i_dim), lambda i, nbe: (nbe[i], 0, 0)),
            pl.BlockSpec((_BM, 1), lambda i, nbe: (i, 0)),
        ],
        out_specs=pl.BlockSpec((_BM, h), lambda i, nbe: (i, 0)),
    )
    return pl.pallas_call(
        _gmm_body,
        grid_spec=grid_spec,
        out_shape=jax.ShapeDtypeStruct((pt, h), jnp.float32),
        compiler_params=pltpu.CompilerParams(
            dimension_semantics=("arbitrary",)),
    )(nbe, xd, Wg, Wu, Wd, w_for_pos.reshape(pt, 1))


def _shared_body(x_ref, sg_ref, su_ref, sd_ref, y_ref):
    xb = x_ref[...].astype(jnp.bfloat16)
    g = lax.dot_general(xb, sg_ref[...].astype(jnp.bfloat16),
                        (((1,), (1,)), ((), ())),
                        preferred_element_type=jnp.float32)
    u = lax.dot_general(xb, su_ref[...].astype(jnp.bfloat16),
                        (((1,), (1,)), ((), ())),
                        preferred_element_type=jnp.float32)
    a = (g * jax.nn.sigmoid(g) * u).astype(jnp.bfloat16)
    y_ref[...] = lax.dot_general(a, sd_ref[...].astype(jnp.bfloat16),
                                 (((1,), (1,)), ((), ())),
                                 preferred_element_type=jnp.float32)


def _shared(x, Sg, Su, Sd):
    t, h = x.shape
    i_dim = Sg.shape[0]
    return pl.pallas_call(
        _shared_body,
        grid=(t // _BT,),
        in_specs=[
            pl.BlockSpec((_BT, h), lambda i: (i, 0)),
            pl.BlockSpec((i_dim, h), lambda i: (0, 0)),
            pl.BlockSpec((i_dim, h), lambda i: (0, 0)),
            pl.BlockSpec((h, i_dim), lambda i: (0, 0)),
        ],
        out_specs=pl.BlockSpec((_BT, h), lambda i: (i, 0)),
        out_shape=jax.ShapeDtypeStruct((t, h), jnp.float32),
    )(x, Sg, Su, Sd)


def _add3_body(a_ref, b_ref, c_ref, y_ref):
    y_ref[...] = a_ref[...] + b_ref[...] + c_ref[...]


def _add3(ya, yb, ysh):
    t, h = ysh.shape
    return pl.pallas_call(
        _add3_body,
        grid=(t // _BT,),
        in_specs=[pl.BlockSpec((_BT, h), lambda i: (i, 0))] * 3,
        out_specs=pl.BlockSpec((_BT, h), lambda i: (i, 0)),
        out_shape=jax.ShapeDtypeStruct((t, h), jnp.float32),
    )(ya, yb, ysh)


def _combine_gather(ye, pa, pb, t):
    h = ye.shape[1]
    info = plsc.get_sparse_core_info()
    nw = info.num_cores * info.num_subcores
    rows_per_w = t // nw
    ch = 16
    n_ch = rows_per_w // ch
    mesh = plsc.VectorSubcoreMesh(core_axis_name="c", subcore_axis_name="s")

    @functools.partial(
        pl.kernel, mesh=mesh,
        out_type=(jax.ShapeDtypeStruct((t, h), jnp.float32),
                  jax.ShapeDtypeStruct((t, h), jnp.float32)),
        scratch_types=[
            pltpu.VMEM((2, ch), jnp.int32),
            pltpu.VMEM((2, ch), jnp.int32),
            pltpu.VMEM((ch, h), jnp.float32),
            pltpu.VMEM((ch, h), jnp.float32),
            pltpu.VMEM((ch, h), jnp.float32),
            pltpu.VMEM((ch, h), jnp.float32),
            pltpu.SemaphoreType.DMA,
            pltpu.SemaphoreType.DMA,
            pltpu.SemaphoreType.DMA,
            pltpu.SemaphoreType.DMA,
        ],
    )
    def k(ye_hbm, pa_hbm, pb_hbm, ya_hbm, yb_hbm,
          ia_v, ib_v, ra0, ra1, rb0, rb1, sa0, sa1, sb0, sb1):
        wid = lax.axis_index("s") * info.num_cores + lax.axis_index("c")
        base = wid * rows_per_w
        ra = (ra0, ra1)
        rb = (rb0, rb1)
        sem_a = (sa0, sa1)
        sem_b = (sb0, sb1)
        d_a = [None, None]
        d_b = [None, None]

        def start(c, b):
            off = base + c * ch
            pltpu.sync_copy(pa_hbm.at[pl.ds(off, ch)], ia_v.at[b])
            pltpu.sync_copy(pb_hbm.at[pl.ds(off, ch)], ib_v.at[b])
            d_a[b] = pltpu.async_copy(ye_hbm.at[ia_v.at[b]], ra[b], sem_a[b])
            d_b[b] = pltpu.async_copy(ye_hbm.at[ib_v.at[b]], rb[b], sem_b[b])

        start(0, 0)
        for c in range(n_ch):
            b = c % 2
            if c + 1 < n_ch:
                start(c + 1, 1 - b)
            d_a[b].wait()
            d_b[b].wait()
            off = base + c * ch
            pltpu.sync_copy(ra[b], ya_hbm.at[pl.ds(off, ch)])
            pltpu.sync_copy(rb[b], yb_hbm.at[pl.ds(off, ch)])

    return k(ye, pa, pb)


def kernel(hidden_states, gate_w, bias, Wg, Wu, Wd, Sg, Su, Sd):
    orig_shape = hidden_states.shape
    x = hidden_states.reshape(-1, orig_shape[-1])
    t = x.shape[0]
    logits = x.astype(jnp.float32) @ gate_w.astype(jnp.float32).T
    comb = _router(logits, bias)
    tok_for_pos, w_for_pos, nbe, pa, pb, pt = _dispatch_indices(comb, t)
    ysh = _shared(x, Sg, Su, Sd)
    xd = _dispatch(x, tok_for_pos, pt)
    ye = _gmm(xd, Wg, Wu, Wd, w_for_pos, nbe, pt)
    ya, yb = _combine_gather(ye, pa, pb, t)
    out = _add3(ya, yb, ysh)
    return out.reshape(orig_shape)


# speedup vs baseline: 1.0032x; 1.0032x over previous
import functools

import jax
import jax.numpy as jnp
from jax import lax
from jax.experimental import pallas as pl
from jax.experimental.pallas import tpu as pltpu
from jax.experimental.pallas import tpu_sc as plsc

_E = 8
_NG = 4
_TK = 2
_RSF = 2.5
_BM = 256
_BT = 512

_NEG_INF = float("-inf")


def _router_body(lg_ref, b_ref, comb_ref):
    logits = lg_ref[...]
    s = jax.nn.sigmoid(logits)
    sc = s + b_ref[...]

    bt = logits.shape[0]
    gs = jnp.concatenate(
        [sc[:, 2 * g:2 * g + 1] + sc[:, 2 * g + 1:2 * g + 2]
         for g in range(_NG)], axis=1)

    i4 = lax.broadcasted_iota(jnp.int32, (bt, _NG), 1)
    m1 = jnp.max(gs, axis=1, keepdims=True)
    g1 = jnp.min(jnp.where(gs == m1, i4, _NG), axis=1, keepdims=True)
    gs2 = jnp.where(i4 == g1, _NEG_INF, gs)
    m2 = jnp.max(gs2, axis=1, keepdims=True)
    g2 = jnp.min(jnp.where(gs2 == m2, i4, _NG), axis=1, keepdims=True)
    gmask = ((i4 == g1) | (i4 == g2)).astype(jnp.float32)
    smask = jnp.concatenate(
        [gmask[:, e // 2:e // 2 + 1] for e in range(_E)], axis=1)

    sfc = jnp.where(smask > 0, sc, 0.0)
    i8 = lax.broadcasted_iota(jnp.int32, (bt, _E), 1)
    e_m1 = jnp.max(sfc, axis=1, keepdims=True)
    e1 = jnp.min(jnp.where(sfc == e_m1, i8, _E), axis=1, keepdims=True)
    sfc2 = jnp.where(i8 == e1, _NEG_INF, sfc)
    e_m2 = jnp.max(sfc2, axis=1, keepdims=True)
    e2 = jnp.min(jnp.where(sfc2 == e_m2, i8, _E), axis=1, keepdims=True)

    sel1 = (i8 == e1)
    sel2 = (i8 == e2)
    w1 = jnp.sum(jnp.where(sel1, s, 0.0), axis=1, keepdims=True)
    w2 = jnp.sum(jnp.where(sel2, s, 0.0), axis=1, keepdims=True)
    scale = _RSF / (w1 + w2 + 1e-20)
    comb_ref[...] = (jnp.where(sel1, w1, 0.0) + jnp.where(sel2, w2, 0.0)) * scale


def _router(logits, bias):
    t = logits.shape[0]
    return pl.pallas_call(
        _router_body,
        grid=(t // _BT,),
        in_specs=[
            pl.BlockSpec((_BT, _E), lambda i: (i, 0)),
            pl.BlockSpec((1, _E), lambda i: (0, 0)),
        ],
        out_specs=pl.BlockSpec((_BT, _E), lambda i: (i, 0)),
        out_shape=jax.ShapeDtypeStruct((t, _E), jnp.float32),
    )(logits, bias.reshape(1, _E))


def _dispatch_indices(comb, t):
    mask = comb > 0.0
    mi = mask.astype(jnp.int32)
    cnt = jnp.sum(mi, axis=0)
    rank = jnp.cumsum(mi, axis=0) - mi
    padded = ((cnt + _BM - 1) // _BM) * _BM
    base = jnp.concatenate([jnp.zeros((1,), jnp.int32),
                            jnp.cumsum(padded)[:-1].astype(jnp.int32)])
    pt = _TK * t + _E * _BM
    pos = base[None, :] + rank
    posm = jnp.where(mask, pos, pt)
    tok = jnp.broadcast_to(jnp.arange(t, dtype=jnp.int32)[:, None], (t, _E))
    tok_for_pos = (jnp.arange(pt + 1, dtype=jnp.int32) % t).at[
        posm.reshape(-1)].set(tok.reshape(-1))[:pt]
    w_for_pos = jnp.zeros((pt + 1,), jnp.float32).at[posm.reshape(-1)].set(
        comb.reshape(-1))[:pt]
    bstart = base // _BM
    blk = jnp.arange(pt // _BM, dtype=jnp.int32)
    nbe = jnp.sum((blk[:, None] >= bstart[None, :]).astype(jnp.int32),
                  axis=1) - 1
    e1 = jnp.argmax(mi, axis=1)
    e2 = (_E - 1) - jnp.argmax(mi[:, ::-1], axis=1)
    ar = jnp.arange(t)
    pa = pos[ar, e1].astype(jnp.int32)
    pb = pos[ar, e2].astype(jnp.int32)
    return tok_for_pos, w_for_pos, nbe, pa, pb, pt


def _dispatch(x, tok_for_pos, pt):
    t, h = x.shape
    info = plsc.get_sparse_core_info()
    nw = info.num_cores * info.num_subcores
    rows_per_w = pt // nw
    ch = 16
    nbuf = 4
    n_ch = rows_per_w // ch
    mesh = plsc.VectorSubcoreMesh(core_axis_name="c", subcore_axis_name="s")

    @functools.partial(
        pl.kernel, mesh=mesh,
        out_type=jax.ShapeDtypeStruct((pt, h), jnp.float32),
        scratch_types=[
            pltpu.VMEM((nbuf, ch), jnp.int32),
            [pltpu.VMEM((ch, h), jnp.float32) for _ in range(nbuf)],
            [pltpu.SemaphoreType.DMA for _ in range(nbuf)],
        ],
    )
    def k(x_hbm, tok_hbm, out_hbm, idx_v, rows, sems):
        wid = lax.axis_index("s") * info.num_cores + lax.axis_index("c")
        base = wid * rows_per_w
        d = [None] * nbuf

        def start(c, b):
            pltpu.sync_copy(tok_hbm.at[pl.ds(base + c * ch, ch)], idx_v.at[b])
            d[b] = pltpu.async_copy(x_hbm.at[idx_v.at[b]], rows[b], sems[b])

        for b in range(nbuf - 1):
            start(b, b)
        for c in range(n_ch):
            b = c % nbuf
            if c + nbuf - 1 < n_ch:
                start(c + nbuf - 1, (c + nbuf - 1) % nbuf)
            d[b].wait()
            pltpu.sync_copy(rows[b], out_hbm.at[pl.ds(base + c * ch, ch)])

    return k(x, tok_for_pos)


def _gmm_body(nbe_ref, xd_ref, wg_ref, wu_ref, wd_ref, w_ref, ye_ref):
    xb = xd_ref[...].astype(jnp.bfloat16)
    wg = wg_ref[0].astype(jnp.bfloat16)
    wu = wu_ref[0].astype(jnp.bfloat16)
    g = lax.dot_general(xb, wg, (((1,), (1,)), ((), ())),
                        preferred_element_type=jnp.float32)
    u = lax.dot_general(xb, wu, (((1,), (1,)), ((), ())),
                        preferred_element_type=jnp.float32)
    a = (g * jax.nn.sigmoid(g) * u).astype(jnp.bfloat16)
    y = lax.dot_general(a, wd_ref[0].astype(jnp.bfloat16),
                        (((1,), (1,)), ((), ())),
                        preferred_element_type=jnp.float32)
    ye_ref[...] = y * w_ref[...]


def _gmm(xd, Wg, Wu, Wd, w_for_pos, nbe, pt):
    h = xd.shape[1]
    i_dim = Wg.shape[1]
    nblk = pt // _BM
    grid_spec = pltpu.PrefetchScalarGridSpec(
        num_scalar_prefetch=1,
        grid=(nblk,),
        in_specs=[
            pl.BlockSpec((_BM, h), lambda i, nbe: (i, 0)),
            pl.BlockSpec((1, i_dim, h), lambda i, nbe: (nbe[i], 0, 0)),
            pl.BlockSpec((1, i_dim, h), lambda i, nbe: (nbe[i], 0, 0)),
            pl.BlockSpec((1, h, i_dim), lambda i, nbe: (nbe[i], 0, 0)),
            pl.BlockSpec((_BM, 1), lambda i, nbe: (i, 0)),
        ],
        out_specs=pl.BlockSpec((_BM, h), lambda i, nbe: (i, 0)),
    )
    return pl.pallas_call(
        _gmm_body,
        grid_spec=grid_spec,
        out_shape=jax.ShapeDtypeStruct((pt, h), jnp.float32),
        compiler_params=pltpu.CompilerParams(
            dimension_semantics=("arbitrary",)),
    )(nbe, xd, Wg, Wu, Wd, w_for_pos.reshape(pt, 1))


def _shared_body(x_ref, sg_ref, su_ref, sd_ref, ya_ref, yb_ref, y_ref):
    xb = x_ref[...].astype(jnp.bfloat16)
    g = lax.dot_general(xb, sg_ref[...].astype(jnp.bfloat16),
                        (((1,), (1,)), ((), ())),
                        preferred_element_type=jnp.float32)
    u = lax.dot_general(xb, su_ref[...].astype(jnp.bfloat16),
                        (((1,), (1,)), ((), ())),
                        preferred_element_type=jnp.float32)
    a = (g * jax.nn.sigmoid(g) * u).astype(jnp.bfloat16)
    sh = lax.dot_general(a, sd_ref[...].astype(jnp.bfloat16),
                         (((1,), (1,)), ((), ())),
                         preferred_element_type=jnp.float32)
    y_ref[...] = sh + ya_ref[...] + yb_ref[...]


def _shared_combine(x, Sg, Su, Sd, ya, yb):
    t, h = x.shape
    i_dim = Sg.shape[0]
    return pl.pallas_call(
        _shared_body,
        grid=(t // _BT,),
        in_specs=[
            pl.BlockSpec((_BT, h), lambda i: (i, 0)),
            pl.BlockSpec((i_dim, h), lambda i: (0, 0)),
            pl.BlockSpec((i_dim, h), lambda i: (0, 0)),
            pl.BlockSpec((h, i_dim), lambda i: (0, 0)),
            pl.BlockSpec((_BT, h), lambda i: (i, 0)),
            pl.BlockSpec((_BT, h), lambda i: (i, 0)),
        ],
        out_specs=pl.BlockSpec((_BT, h), lambda i: (i, 0)),
        out_shape=jax.ShapeDtypeStruct((t, h), jnp.float32),
    )(x, Sg, Su, Sd, ya, yb)


def _combine_gather(ye, pa, pb, t):
    h = ye.shape[1]
    info = plsc.get_sparse_core_info()
    nw = info.num_cores * info.num_subcores
    rows_per_w = t // nw
    ch = 16
    n_ch = rows_per_w // ch
    mesh = plsc.VectorSubcoreMesh(core_axis_name="c", subcore_axis_name="s")

    @functools.partial(
        pl.kernel, mesh=mesh,
        out_type=(jax.ShapeDtypeStruct((t, h), jnp.float32),
                  jax.ShapeDtypeStruct((t, h), jnp.float32)),
        scratch_types=[
            pltpu.VMEM((2, ch), jnp.int32),
            pltpu.VMEM((2, ch), jnp.int32),
            pltpu.VMEM((ch, h), jnp.float32),
            pltpu.VMEM((ch, h), jnp.float32),
            pltpu.VMEM((ch, h), jnp.float32),
            pltpu.VMEM((ch, h), jnp.float32),
            pltpu.SemaphoreType.DMA,
            pltpu.SemaphoreType.DMA,
            pltpu.SemaphoreType.DMA,
            pltpu.SemaphoreType.DMA,
        ],
    )
    def k(ye_hbm, pa_hbm, pb_hbm, ya_hbm, yb_hbm,
          ia_v, ib_v, ra0, ra1, rb0, rb1, sa0, sa1, sb0, sb1):
        wid = lax.axis_index("s") * info.num_cores + lax.axis_index("c")
        base = wid * rows_per_w
        ra = (ra0, ra1)
        rb = (rb0, rb1)
        sem_a = (sa0, sa1)
        sem_b = (sb0, sb1)
        d_a = [None, None]
        d_b = [None, None]

        def start(c, b):
            off = base + c * ch
            pltpu.sync_copy(pa_hbm.at[pl.ds(off, ch)], ia_v.at[b])
            pltpu.sync_copy(pb_hbm.at[pl.ds(off, ch)], ib_v.at[b])
            d_a[b] = pltpu.async_copy(ye_hbm.at[ia_v.at[b]], ra[b], sem_a[b])
            d_b[b] = pltpu.async_copy(ye_hbm.at[ib_v.at[b]], rb[b], sem_b[b])

        start(0, 0)
        for c in range(n_ch):
            b = c % 2
            if c + 1 < n_ch:
                start(c + 1, 1 - b)
            d_a[b].wait()
            d_b[b].wait()
            off = base + c * ch
            pltpu.sync_copy(ra[b], ya_hbm.at[pl.ds(off, ch)])
            pltpu.sync_copy(rb[b], yb_hbm.at[pl.ds(off, ch)])

    return k(ye, pa, pb)


def kernel(hidden_states, gate_w, bias, Wg, Wu, Wd, Sg, Su, Sd):
    orig_shape = hidden_states.shape
    x = hidden_states.reshape(-1, orig_shape[-1])
    t = x.shape[0]
    logits = x.astype(jnp.float32) @ gate_w.astype(jnp.float32).T
    comb = _router(logits, bias)
    tok_for_pos, w_for_pos, nbe, pa, pb, pt = _dispatch_indices(comb, t)
    xd = _dispatch(x, tok_for_pos, pt)
    ye = _gmm(xd, Wg, Wu, Wd, w_for_pos, nbe, pt)
    ya, yb = _combine_gather(ye, pa, pb, t)
    out = _shared_combine(x, Sg, Su, Sd, ya, yb)
    return out.reshape(orig_shape)
